# baseline (speedup 1.0000x reference)
import functools

import jax
import jax.numpy as jnp
import numpy as np
from jax import lax
from jax.experimental import pallas as pl
from jax.experimental.pallas import tpu as pltpu
from jax.experimental.pallas import tpu_sc as plsc

VOXEL_SIZE = 0.04

_NW = 32
_V = 9


def _block_points(n_cols):
    return 128 if n_cols <= 32 else 64


@functools.partial(jax.jit, static_argnames=("n_pts", "n_cols"))
def _sc_gather_viewsum(table, idx4, n_pts, n_cols):
    IBp = _block_points(n_cols)
    per_w = n_pts // _NW
    nblk = per_w // IBp
    mesh = plsc.VectorSubcoreMesh(core_axis_name="c", subcore_axis_name="s")

    @functools.partial(
        pl.kernel,
        mesh=mesh,
        out_type=jax.ShapeDtypeStruct((_NW, nblk, IBp, n_cols), jnp.float32),
        compiler_params=pltpu.CompilerParams(use_tc_tiling_on_sc=False),
        scratch_types=[
            pltpu.VMEM((2, _V, IBp), jnp.int32),
            pltpu.VMEM((2, _V, IBp, n_cols), jnp.float32),
            pltpu.VMEM((2, IBp, n_cols), jnp.float32),
            [pltpu.SemaphoreType.DMA] * 2,
            [pltpu.SemaphoreType.DMA] * 2,
            [pltpu.SemaphoreType.DMA] * 2,
        ],
    )
    def gather_kernel(table_hbm, idx_hbm, out_hbm, idx_v, rows_v, acc_v,
                      isem, gsem, osem):
        cid = lax.axis_index("c")
        sid = lax.axis_index("s")
        wid = sid * 2 + cid

        def idx_copy(b, s):
            return pltpu.make_async_copy(idx_hbm.at[wid, b], idx_v.at[s], isem[s])

        def out_copy(b, s):
            return pltpu.make_async_copy(acc_v.at[s], out_hbm.at[wid, b], osem[s])

        def view_gather(s, v):
            return pltpu.make_async_copy(
                table_hbm.at[idx_v.at[s, v]], rows_v.at[s, v], gsem[s])

        idx_copy(0, 0).start()
        idx_copy(1, 1).start()

        def step(b, s):
            idx_copy(b, s).wait()
            for v in range(_V):
                view_gather(s, v).start()
            for v in range(_V):
                view_gather(s, v).wait()

            @pl.when(b + 2 < nblk)
            def _():
                idx_copy(b + 2, s).start()

            @pl.when(b >= 2)
            def _():
                out_copy(b, s).wait()

            def acc_body(p, carry):
                for c0 in range(0, n_cols, 16):
                    acc = rows_v[s, 0, p, pl.ds(c0, 16)]
                    for v in range(1, _V):
                        acc = acc + rows_v[s, v, p, pl.ds(c0, 16)]
                    acc_v[s, p, pl.ds(c0, 16)] = acc
                return carry

            lax.fori_loop(0, IBp, acc_body, 0)
            out_copy(b, s).start()

        def pair_body(q, carry):
            step(q * 2, 0)
            step(q * 2 + 1, 1)
            return carry

        lax.fori_loop(0, nblk // 2, pair_body, 0)
        out_copy(nblk - 2, 0).wait()
        out_copy(nblk - 1, 1).wait()

    return gather_kernel(table, idx4)


def _back_project_padded(coords, vol_origin, feats, KR):
    world = coords * VOXEL_SIZE + vol_origin[None, :]
    homog = jnp.concatenate([world, jnp.ones_like(world[:, :1])], axis=1)
    cam = jnp.einsum("vij,nj->vni", KR, homog)
    z = cam[..., 2]
    zs = jnp.maximum(z, 1e-6)
    px = cam[..., 0] / zs
    py = cam[..., 1] / zs
    V, C, H, W = feats.shape
    mask = (z > 0.1) & (px >= 0) & (px <= W - 1) & (py >= 0) & (py <= H - 1)
    ix = jnp.clip(jnp.round(px).astype(jnp.int32), 0, W - 1)
    iy = jnp.clip(jnp.round(py).astype(jnp.int32), 0, H - 1)
    lin = iy * W + ix

    Cp = ((C + 15) // 16) * 16
    table = feats.transpose(0, 2, 3, 1).reshape(V * H * W, C)
    if Cp != C:
        table = jnp.pad(table, ((0, 0), (0, Cp - C)))
    zero_row = V * H * W
    table = jnp.pad(table, ((0, 1), (0, 0)))

    N = coords.shape[0]
    offs = (jnp.arange(V, dtype=jnp.int32) * (H * W))[:, None]
    idx = jnp.where(mask, lin + offs, zero_row)

    IBp = _block_points(Cp)
    gran = _NW * 2 * IBp
    Np = ((N + gran - 1) // gran) * gran
    if Np != N:
        idx = jnp.pad(idx, ((0, 0), (0, Np - N)), constant_values=zero_row)
    nblk = Np // (_NW * IBp)
    idx4 = idx.reshape(V, _NW, nblk, IBp).transpose(1, 2, 0, 3)

    vol_sum = _sc_gather_viewsum(table, idx4, Np, Cp).reshape(Np, Cp)[:N, :C]
    count = mask.sum(axis=0).astype(jnp.float32)
    vol = vol_sum / jnp.maximum(count[:, None], 1.0)
    return jnp.concatenate([vol, count[:, None]], axis=1)


def _generate_grid(n_vox, interval):
    r = jnp.arange(0, n_vox, interval, dtype=jnp.float32)
    gx, gy, gz = jnp.meshgrid(r, r, r, indexing="ij")
    return jnp.stack([gx.ravel(), gy.ravel(), gz.ravel()], axis=1)


def _upsample(prev_feat, prev_coords, interval):
    off = np.zeros((8, 3), dtype=np.float32)
    pos_list = [[0], [1], [2], [0, 1], [0, 2], [1, 2], [0, 1, 2]]
    for i, p in enumerate(pos_list):
        off[i + 1, p] = interval
    off = jnp.asarray(off)
    up_coords = (prev_coords[:, None, :] + off[None]).reshape(-1, 3)
    up_feat = jnp.repeat(prev_feat, 8, axis=0)
    return up_feat, up_coords


def kernel(feats_s0, feats_s1, feats_s2, proj_s0, proj_s1, proj_s2, vol_origin,
           W1_0, b1_0, W2_0, b2_0, Wt_0, bt_0, Wo_0, bo_0,
           W1_1, b1_1, W2_1, b2_1, Wt_1, bt_1, Wo_1, bo_1,
           W1_2, b1_2, W2_2, b2_2, Wt_2, bt_2, Wo_2, bo_2):
    p = dict(W1_0=W1_0, b1_0=b1_0, W2_0=W2_0, b2_0=b2_0,
             Wt_0=Wt_0, bt_0=bt_0, Wo_0=Wo_0, bo_0=bo_0,
             W1_1=W1_1, b1_1=b1_1, W2_1=W2_1, b2_1=b2_1,
             Wt_1=Wt_1, bt_1=bt_1, Wo_1=Wo_1, bo_1=bo_1,
             W1_2=W1_2, b1_2=b1_2, W2_2=W2_2, b2_2=b2_2,
             Wt_2=Wt_2, bt_2=bt_2, Wo_2=Wo_2, bo_2=bo_2)
    feats_all = [feats_s0, feats_s1, feats_s2]
    proj_all = [proj_s0, proj_s1, proj_s2]
    n_scales = 2
    prev_feat = None
    prev_coords = None
    out = None
    for i in range(3):
        interval = 2 ** (n_scales - i)
        scale = n_scales - i
        if i == 0:
            up_coords = _generate_grid(96, interval)
        else:
            up_feat, up_coords = _upsample(prev_feat, prev_coords, interval)
        volume = _back_project_padded(
            up_coords, vol_origin, feats_all[scale], proj_all[scale])
        if i == 0:
            feat = volume
        else:
            feat = jnp.concatenate([volume, up_feat], axis=1)
        h = jax.nn.relu(feat @ p["W1_%d" % i] + p["b1_%d" % i])
        h = jax.nn.relu(h @ p["W2_%d" % i] + p["b2_%d" % i])
        tsdf = h @ p["Wt_%d" % i] + p["bt_%d" % i]
        occ = h @ p["Wo_%d" % i] + p["bo_%d" % i]
        kkeep = h.shape[0] // 2
        _, idx = jax.lax.top_k(occ[:, 0], kkeep)
        prev_coords = jnp.take(up_coords, idx, axis=0)
        prev_tsdf = jnp.take(tsdf, idx, axis=0)
        prev_occ = jnp.take(occ, idx, axis=0)
        keep_h = jnp.take(h, idx, axis=0)
        prev_feat = jnp.concatenate([keep_h, prev_tsdf, prev_occ], axis=1)
        if i == 2:
            out = (prev_coords, prev_tsdf)
    return out
